# baseline (speedup 1.0000x reference)
import dataclasses
import functools

import jax
import jax.numpy as jnp
from jax import lax
from jax.experimental import pallas as pl
from jax.experimental.pallas import tpu as pltpu
from jax.experimental.pallas import tpu_sc as plsc

N = 10000
E = 320000
E_TOT = E + N
F_IN = 128
D = 64
NEG_SLOPE = 0.2

NUM_CORES = 2
NUM_SUBCORES = 16
NUM_TILES = NUM_CORES * NUM_SUBCORES
PT = 10368
E_PAD = NUM_TILES * PT
W = 192
N_WIN = PT // W
N_GRP = W // 16
NP = 10240
RPS = NP // NUM_SUBCORES


def _edge_pass_body(src_hbm, dst_hbm, xl_hbm, xr_hbm, att_hbm, z64_hbm,
                    z1_hbm, num_out, den_out,
                    srcv0, srcv1, dstv0, dstv1, dsc0, dsc1, p0, p1,
                    xl0, xl1, xr0, xr1, sc0, sc1, att_v,
                    num_shared, den_shared,
                    sem_g0, sem_g1, sem_s0, sem_s1):
    cid = lax.axis_index("c")
    sid = lax.axis_index("s")
    wid = sid * NUM_CORES + cid
    r0 = sid * RPS

    pltpu.sync_copy(z64_hbm.at[pl.ds(r0, RPS)], num_shared.at[pl.ds(r0, RPS)])

    @pl.when(sid == 0)
    def _():
        pltpu.sync_copy(z1_hbm, den_shared)

    pltpu.sync_copy(att_hbm, att_v)
    att_vecs = [att_v[pl.ds(k * 16, 16)] for k in range(D // 16)]
    plsc.subcore_barrier()

    stages = ((srcv0, dstv0, dsc0, p0, xl0, xr0, sc0, sem_g0, sem_s0),
              (srcv1, dstv1, dsc1, p1, xl1, xr1, sc1, sem_g1, sem_s1))

    def issue_gather(w, st):
        srcv, dstv, dsc, p_v, xlr, xrr, scr, sg, ss = st
        base = wid * PT + w * W
        pltpu.sync_copy(src_hbm.at[pl.ds(base, W)], srcv)
        pltpu.sync_copy(dst_hbm.at[pl.ds(base, W)], dstv)
        pltpu.async_copy(xl_hbm.at[srcv], xlr, sg)
        pltpu.async_copy(xr_hbm.at[dstv], xrr, sg)

    def wait_gather(st):
        srcv, dstv, dsc, p_v, xlr, xrr, scr, sg, ss = st
        pltpu.make_async_copy(xl_hbm.at[srcv], xlr, sg).wait()
        pltpu.make_async_copy(xr_hbm.at[dstv], xrr, sg).wait()

    def wait_scatter(st):
        srcv, dstv, dsc, p_v, xlr, xrr, scr, sg, ss = st
        pltpu.make_async_copy(p_v, den_shared.at[dsc], ss).wait()
        pltpu.make_async_copy(scr, num_shared.at[dsc], ss).wait()

    def stage_body(w, st):
        srcv, dstv, dsc, p_v, xlr, xrr, scr, sg, ss = st
        base = wid * PT + w * W
        wait_gather(st)

        @pl.when(w >= 2)
        def _():
            wait_scatter(st)

        for k in range(N_GRP):
            dsc[pl.ds(k * 16, 16)] = dstv[pl.ds(k * 16, 16)]

        lane0 = lax.iota(jnp.int32, 16) == 0

        @plsc.parallel_loop(0, W, step=1, unroll=16)
        def _edge(row):
            vls = []
            tot = None
            for k in range(D // 16):
                vl = xlr[row, pl.ds(k * 16, 16)]
                vr = xrr[row, pl.ds(k * 16, 16)]
                vls.append(vl)
                s = vl + vr
                t = att_vecs[k] * jnp.maximum(s, NEG_SLOPE * s)
                tot = t if tot is None else tot + t
            logit = jnp.sum(tot)
            pvec = jnp.exp(jnp.full((16,), logit, jnp.float32))
            pvec = jnp.where(base + row < E_TOT, pvec,
                             jnp.zeros((16,), jnp.float32))
            plsc.store_scatter(p_v, [jnp.full((16,), row, jnp.int32)],
                               pvec, mask=lane0)
            for k in range(D // 16):
                scr[row, pl.ds(k * 16, 16)] = vls[k] * pvec

        pltpu.async_copy(p_v, den_shared.at[dsc], ss, add=True)
        pltpu.async_copy(scr, num_shared.at[dsc], ss, add=True)

        @pl.when(w + 2 < N_WIN)
        def _():
            issue_gather(w + 2, st)

    issue_gather(0, stages[0])
    issue_gather(1, stages[1])

    @pl.loop(0, N_WIN, step=2)
    def _win(w):
        stage_body(w, stages[0])
        stage_body(w + 1, stages[1])

    wait_scatter(stages[0])
    wait_scatter(stages[1])
    plsc.subcore_barrier()
    pltpu.sync_copy(num_shared.at[pl.ds(r0, RPS)],
                    num_out.at[cid].at[pl.ds(r0, RPS)])

    @pl.when(sid == 0)
    def _():
        pltpu.sync_copy(den_shared, den_out.at[cid])


def _edge_pass(src, dst, xl, xr, att, z64, z1):
    mesh = plsc.VectorSubcoreMesh(core_axis_name="c", subcore_axis_name="s")
    cp = pltpu.CompilerParams(needs_layout_passes=False,
                              use_tc_tiling_on_sc=False)
    fn = pl.kernel(
        _edge_pass_body,
        mesh=mesh,
        compiler_params=cp,
        out_type=(jax.ShapeDtypeStruct((NUM_CORES, NP, D), jnp.float32),
                  jax.ShapeDtypeStruct((NUM_CORES, NP), jnp.float32)),
        scratch_types=(
            [pltpu.VMEM((W,), jnp.int32) for _ in range(6)]
            + [pltpu.VMEM((W,), jnp.float32) for _ in range(2)]
            + [pltpu.VMEM((W, D), jnp.float32) for _ in range(6)]
            + [pltpu.VMEM((D,), jnp.float32),
               pltpu.VMEM_SHARED((NP, D), jnp.float32),
               pltpu.VMEM_SHARED((NP,), jnp.float32),
               pltpu.SemaphoreType.DMA, pltpu.SemaphoreType.DMA,
               pltpu.SemaphoreType.DMA, pltpu.SemaphoreType.DMA]
        ),
    )
    return fn(src, dst, xl, xr, att, z64, z1)


_B = 2000


def _transform1_body(x_ref, wl_ref, bl_ref, wr_ref, br_ref, xl_ref, xr_ref):
    xb = x_ref[...]
    dn = (((1,), (1,)), ((), ()))
    xl_ref[...] = lax.dot_general(xb, wl_ref[...], dn,
                                  preferred_element_type=jnp.float32) + bl_ref[...]
    xr_ref[...] = lax.dot_general(xb, wr_ref[...], dn,
                                  preferred_element_type=jnp.float32) + br_ref[...]


def _transform1(x, Wl, bl, Wr, br):
    grid = (N // _B,)
    return pl.pallas_call(
        _transform1_body,
        grid=grid,
        in_specs=[
            pl.BlockSpec((_B, F_IN), lambda i: (i, 0)),
            pl.BlockSpec((D, F_IN), lambda i: (0, 0)),
            pl.BlockSpec((D,), lambda i: (0,)),
            pl.BlockSpec((D, F_IN), lambda i: (0, 0)),
            pl.BlockSpec((D,), lambda i: (0,)),
        ],
        out_specs=(pl.BlockSpec((_B, D), lambda i: (i, 0)),
                   pl.BlockSpec((_B, D), lambda i: (i, 0))),
        out_shape=(jax.ShapeDtypeStruct((N, D), jnp.float32),
                   jax.ShapeDtypeStruct((N, D), jnp.float32)),
    )(x, Wl, bl, Wr, br)


def _combine2_body(num_ref, den_ref, b1_ref, wl_ref, bl_ref, wr_ref, br_ref,
                   xl_ref, xr_ref):
    nsum = num_ref[0] + num_ref[1]
    h = jnp.maximum(nsum / den_ref[...] + b1_ref[...], 0.0)
    dn = (((1,), (1,)), ((), ()))
    xl_ref[...] = lax.dot_general(h, wl_ref[...], dn,
                                  preferred_element_type=jnp.float32) + bl_ref[...]
    xr_ref[...] = lax.dot_general(h, wr_ref[...], dn,
                                  preferred_element_type=jnp.float32) + br_ref[...]


def _combine_transform(num_p, den, b1, Wl, bl, Wr, br):
    grid = (N // _B,)
    return pl.pallas_call(
        _combine2_body,
        grid=grid,
        in_specs=[
            pl.BlockSpec((NUM_CORES, _B, D), lambda i: (0, i, 0)),
            pl.BlockSpec((_B, 1), lambda i: (i, 0)),
            pl.BlockSpec((D,), lambda i: (0,)),
            pl.BlockSpec((D, D), lambda i: (0, 0)),
            pl.BlockSpec((D,), lambda i: (0,)),
            pl.BlockSpec((D, D), lambda i: (0, 0)),
            pl.BlockSpec((D,), lambda i: (0,)),
        ],
        out_specs=(pl.BlockSpec((_B, D), lambda i: (i, 0)),
                   pl.BlockSpec((_B, D), lambda i: (i, 0))),
        out_shape=(jax.ShapeDtypeStruct((N, D), jnp.float32),
                   jax.ShapeDtypeStruct((N, D), jnp.float32)),
    )(num_p, den, b1, Wl, bl, Wr, br)


def _final_body(num_ref, den_ref, b2_ref, out_ref):
    nsum = num_ref[0] + num_ref[1]
    out_ref[...] = nsum[:, :40] / den_ref[...] + b2_ref[...]


def _final(num_p, den, b2):
    grid = (N // _B,)
    return pl.pallas_call(
        _final_body,
        grid=grid,
        in_specs=[
            pl.BlockSpec((NUM_CORES, _B, D), lambda i: (0, i, 0)),
            pl.BlockSpec((_B, 1), lambda i: (i, 0)),
            pl.BlockSpec((40,), lambda i: (0,)),
        ],
        out_specs=pl.BlockSpec((_B, 40), lambda i: (i, 0)),
        out_shape=jax.ShapeDtypeStruct((N, 40), jnp.float32),
    )(num_p, den, b2)


def kernel(x, edge_index, Wl1, bl1, Wr1, br1, att1, bias1,
           Wl2, bl2, Wr2, br2, att2, bias2):
    loops = jnp.arange(N, dtype=edge_index.dtype)
    ei = jnp.concatenate(
        [edge_index, jnp.stack([loops, loops], axis=0)], axis=1)
    ei = jnp.pad(ei, ((0, 0), (0, E_PAD - E_TOT)))
    src = ei[0]
    dst = ei[1]
    z64 = jnp.zeros((NP, D), jnp.float32)
    z1 = jnp.zeros((NP,), jnp.float32)

    xl1, xr1 = _transform1(x, Wl1, bl1, Wr1, br1)
    np1, dp1 = _edge_pass(src, dst, xl1, xr1, att1, z64, z1)
    den1 = (dp1[0, :N] + dp1[1, :N] + 1e-16).reshape(N, 1)

    Wl2p = jnp.zeros((D, D), jnp.float32).at[:40].set(Wl2)
    Wr2p = jnp.zeros((D, D), jnp.float32).at[:40].set(Wr2)
    bl2p = jnp.zeros((D,), jnp.float32).at[:40].set(bl2)
    br2p = jnp.zeros((D,), jnp.float32).at[:40].set(br2)
    att2p = jnp.zeros((D,), jnp.float32).at[:40].set(att2)

    xl2, xr2 = _combine_transform(np1, den1, bias1, Wl2p, bl2p, Wr2p, br2p)
    np2, dp2 = _edge_pass(src, dst, xl2, xr2, att2p, z64, z1)
    den2 = (dp2[0, :N] + dp2[1, :N] + 1e-16).reshape(N, 1)
    out = _final(np2, den2, bias2)
    return (out, edge_index)


# speedup vs baseline: 1.0926x; 1.0926x over previous
import dataclasses
import functools

import numpy as np

import jax
import jax.numpy as jnp
from jax import lax
from jax.experimental import pallas as pl
from jax.experimental.pallas import tpu as pltpu
from jax.experimental.pallas import tpu_sc as plsc

N = 10000
E = 320000
E_TOT = E + N
F_IN = 128
D = 64
NEG_SLOPE = 0.2

NUM_CORES = 2
NUM_SUBCORES = 16
NUM_TILES = NUM_CORES * NUM_SUBCORES
PT = 10368
E_PAD = NUM_TILES * PT
W = 288
N_WIN = PT // W
N_GRP = W // 16
NP = 10240
RPS = NP // NUM_SUBCORES

_IDX = np.empty((D,), np.int32)
for _k in range(D // 32):
    _IDX[32 * _k:32 * _k + 32:2] = np.arange(32 * _k, 32 * _k + 16)
    _IDX[32 * _k + 1:32 * _k + 32:2] = np.arange(32 * _k + 16, 32 * _k + 32)
IDX = tuple(int(i) for i in _IDX)


def _edge_pass_body(src_hbm, dst_hbm, xl_hbm, xr_hbm, att_hbm, z64_hbm,
                    z1_hbm, num_out, den_out,
                    srcv0, srcv1, dstv0, dstv1, dsc0, dsc1, p0, p1,
                    xl0, xl1, xr0, xr1, sc0, sc1, att_v,
                    num_shared, den_shared,
                    sem_g0, sem_g1, sem_s0, sem_s1):
    cid = lax.axis_index("c")
    sid = lax.axis_index("s")
    wid = sid * NUM_CORES + cid
    r0 = sid * RPS

    pltpu.sync_copy(z64_hbm.at[pl.ds(r0, RPS)], num_shared.at[pl.ds(r0, RPS)])

    @pl.when(sid == 0)
    def _():
        pltpu.sync_copy(z1_hbm, den_shared)

    pltpu.sync_copy(att_hbm, att_v)
    att_vecs = [att_v[pl.ds(k * 16, 16)] for k in range(D // 16)]
    plsc.subcore_barrier()

    stages = ((srcv0, dstv0, dsc0, p0, xl0, xr0, sc0, sem_g0, sem_s0),
              (srcv1, dstv1, dsc1, p1, xl1, xr1, sc1, sem_g1, sem_s1))

    def issue_gather(w, st):
        srcv, dstv, dsc, p_v, xlr, xrr, scr, sg, ss = st
        base = wid * PT + w * W
        pltpu.sync_copy(src_hbm.at[pl.ds(base, W)], srcv)
        pltpu.sync_copy(dst_hbm.at[pl.ds(base, W)], dstv)
        pltpu.async_copy(xl_hbm.at[srcv], xlr, sg)
        pltpu.async_copy(xr_hbm.at[dstv], xrr, sg)

    def wait_gather(st):
        srcv, dstv, dsc, p_v, xlr, xrr, scr, sg, ss = st
        pltpu.make_async_copy(xl_hbm.at[srcv], xlr, sg).wait()
        pltpu.make_async_copy(xr_hbm.at[dstv], xrr, sg).wait()

    def wait_scatter(st):
        srcv, dstv, dsc, p_v, xlr, xrr, scr, sg, ss = st
        pltpu.make_async_copy(p_v, den_shared.at[dsc], ss).wait()
        pltpu.make_async_copy(scr, num_shared.at[dsc], ss).wait()

    def stage_body(w, st):
        srcv, dstv, dsc, p_v, xlr, xrr, scr, sg, ss = st
        base = wid * PT + w * W
        wait_gather(st)

        @pl.when(w >= 2)
        def _():
            wait_scatter(st)

        for k in range(N_GRP):
            dsc[pl.ds(k * 16, 16)] = dstv[pl.ds(k * 16, 16)]

        lane0 = lax.iota(jnp.int32, 16) == 0

        @plsc.parallel_loop(0, W, step=1, unroll=8)
        def _edge(row):
            vls = []
            tot = None
            for k in range(D // 32):
                vl32 = xlr[row, pl.ds(k * 32, 32)]
                vr32 = xrr[row, pl.ds(k * 32, 32)]
                la, lb = plsc.unpack(vl32, format=plsc.PackFormat.INTERLEAVED,
                                     preferred_element_type=jnp.float32)
                ra, rb = plsc.unpack(vr32, format=plsc.PackFormat.INTERLEAVED,
                                     preferred_element_type=jnp.float32)
                for j, (vl, vr) in enumerate(((la, ra), (lb, rb))):
                    vls.append(vl)
                    s = vl + vr
                    t = att_vecs[2 * k + j] * jnp.maximum(s, NEG_SLOPE * s)
                    tot = t if tot is None else tot + t
            logit = jnp.sum(tot)
            pvec = jnp.exp(jnp.full((16,), logit, jnp.float32))
            pvec = jnp.where(base + row < E_TOT, pvec,
                             jnp.zeros((16,), jnp.float32))
            plsc.store_scatter(p_v, [jnp.full((16,), row, jnp.int32)],
                               pvec, mask=lane0)
            for k in range(D // 16):
                scr[row, pl.ds(k * 16, 16)] = vls[k] * pvec

        pltpu.async_copy(p_v, den_shared.at[dsc], ss, add=True)
        pltpu.async_copy(scr, num_shared.at[dsc], ss, add=True)

        @pl.when(w + 2 < N_WIN)
        def _():
            issue_gather(w + 2, st)

    issue_gather(0, stages[0])
    issue_gather(1, stages[1])

    @pl.loop(0, N_WIN, step=2)
    def _win(w):
        stage_body(w, stages[0])
        stage_body(w + 1, stages[1])

    wait_scatter(stages[0])
    wait_scatter(stages[1])
    plsc.subcore_barrier()
    pltpu.sync_copy(num_shared.at[pl.ds(r0, RPS)],
                    num_out.at[cid].at[pl.ds(r0, RPS)])

    @pl.when(sid == 0)
    def _():
        pltpu.sync_copy(den_shared, den_out.at[cid])


def _edge_pass(src, dst, xl, xr, att, z64, z1):
    mesh = plsc.VectorSubcoreMesh(core_axis_name="c", subcore_axis_name="s")
    cp = pltpu.CompilerParams(needs_layout_passes=False,
                              use_tc_tiling_on_sc=False)
    fn = pl.kernel(
        _edge_pass_body,
        mesh=mesh,
        compiler_params=cp,
        out_type=(jax.ShapeDtypeStruct((NUM_CORES, NP, D), jnp.float32),
                  jax.ShapeDtypeStruct((NUM_CORES, NP), jnp.float32)),
        scratch_types=(
            [pltpu.VMEM((W,), jnp.int32) for _ in range(6)]
            + [pltpu.VMEM((W,), jnp.float32) for _ in range(2)]
            + [pltpu.VMEM((W, D), jnp.bfloat16) for _ in range(4)]
            + [pltpu.VMEM((W, D), jnp.float32) for _ in range(2)]
            + [pltpu.VMEM((D,), jnp.float32),
               pltpu.VMEM_SHARED((NP, D), jnp.float32),
               pltpu.VMEM_SHARED((NP,), jnp.float32),
               pltpu.SemaphoreType.DMA, pltpu.SemaphoreType.DMA,
               pltpu.SemaphoreType.DMA, pltpu.SemaphoreType.DMA]
        ),
    )
    return fn(src, dst, xl, xr, att, z64, z1)


_B = 2000


def _transform1_body(x_ref, wl_ref, bl_ref, wr_ref, br_ref, xl_ref, xr_ref):
    xb = x_ref[...]
    dn = (((1,), (1,)), ((), ()))
    xl_ref[...] = (lax.dot_general(xb, wl_ref[...], dn,
                                   preferred_element_type=jnp.float32)
                   + bl_ref[...]).astype(jnp.bfloat16)
    xr_ref[...] = (lax.dot_general(xb, wr_ref[...], dn,
                                   preferred_element_type=jnp.float32)
                   + br_ref[...]).astype(jnp.bfloat16)


def _transform1(x, Wl, bl, Wr, br):
    grid = (N // _B,)
    return pl.pallas_call(
        _transform1_body,
        grid=grid,
        in_specs=[
            pl.BlockSpec((_B, F_IN), lambda i: (i, 0)),
            pl.BlockSpec((D, F_IN), lambda i: (0, 0)),
            pl.BlockSpec((D,), lambda i: (0,)),
            pl.BlockSpec((D, F_IN), lambda i: (0, 0)),
            pl.BlockSpec((D,), lambda i: (0,)),
        ],
        out_specs=(pl.BlockSpec((_B, D), lambda i: (i, 0)),
                   pl.BlockSpec((_B, D), lambda i: (i, 0))),
        out_shape=(jax.ShapeDtypeStruct((N, D), jnp.bfloat16),
                   jax.ShapeDtypeStruct((N, D), jnp.bfloat16)),
    )(x, Wl, bl, Wr, br)


def _combine2_body(num_ref, den_ref, b1_ref, wl_ref, bl_ref, wr_ref, br_ref,
                   xl_ref, xr_ref):
    nsum = num_ref[0] + num_ref[1]
    h = jnp.maximum(nsum / den_ref[...] + b1_ref[...], 0.0)
    dn = (((1,), (1,)), ((), ()))
    xl_ref[...] = (lax.dot_general(h, wl_ref[...], dn,
                                   preferred_element_type=jnp.float32)
                   + bl_ref[...]).astype(jnp.bfloat16)
    xr_ref[...] = (lax.dot_general(h, wr_ref[...], dn,
                                   preferred_element_type=jnp.float32)
                   + br_ref[...]).astype(jnp.bfloat16)


def _combine_transform(num_p, den, b1, Wl, bl, Wr, br):
    grid = (N // _B,)
    return pl.pallas_call(
        _combine2_body,
        grid=grid,
        in_specs=[
            pl.BlockSpec((NUM_CORES, _B, D), lambda i: (0, i, 0)),
            pl.BlockSpec((_B, 1), lambda i: (i, 0)),
            pl.BlockSpec((D,), lambda i: (0,)),
            pl.BlockSpec((D, D), lambda i: (0, 0)),
            pl.BlockSpec((D,), lambda i: (0,)),
            pl.BlockSpec((D, D), lambda i: (0, 0)),
            pl.BlockSpec((D,), lambda i: (0,)),
        ],
        out_specs=(pl.BlockSpec((_B, D), lambda i: (i, 0)),
                   pl.BlockSpec((_B, D), lambda i: (i, 0))),
        out_shape=(jax.ShapeDtypeStruct((N, D), jnp.bfloat16),
                   jax.ShapeDtypeStruct((N, D), jnp.bfloat16)),
    )(num_p, den, b1, Wl, bl, Wr, br)


def _final_body(num_ref, den_ref, b2_ref, out_ref):
    nsum = num_ref[0] + num_ref[1]
    out_ref[...] = nsum[:, :40] / den_ref[...] + b2_ref[...]


def _final(num_p, den, b2):
    grid = (N // _B,)
    return pl.pallas_call(
        _final_body,
        grid=grid,
        in_specs=[
            pl.BlockSpec((NUM_CORES, _B, D), lambda i: (0, i, 0)),
            pl.BlockSpec((_B, 1), lambda i: (i, 0)),
            pl.BlockSpec((40,), lambda i: (0,)),
        ],
        out_specs=pl.BlockSpec((_B, 40), lambda i: (i, 0)),
        out_shape=jax.ShapeDtypeStruct((N, 40), jnp.float32),
    )(num_p, den, b2)


def kernel(x, edge_index, Wl1, bl1, Wr1, br1, att1, bias1,
           Wl2, bl2, Wr2, br2, att2, bias2):
    loops = jnp.arange(N, dtype=edge_index.dtype)
    ei = jnp.concatenate(
        [edge_index, jnp.stack([loops, loops], axis=0)], axis=1)
    ei = jnp.pad(ei, ((0, 0), (0, E_PAD - E_TOT)))
    src = ei[0]
    dst = ei[1]
    z64 = jnp.zeros((NP, D), jnp.float32)
    z1 = jnp.zeros((NP,), jnp.float32)

    xl1, xr1 = _transform1(x, Wl1[IDX, :], bl1[IDX,], Wr1[IDX, :], br1[IDX,])
    np1, dp1 = _edge_pass(src, dst, xl1, xr1, att1, z64, z1)
    den1 = (dp1[0, :N] + dp1[1, :N] + 1e-16).reshape(N, 1)

    Wl2p = jnp.zeros((D, D), jnp.float32).at[:40].set(Wl2)[IDX, :]
    Wr2p = jnp.zeros((D, D), jnp.float32).at[:40].set(Wr2)[IDX, :]
    bl2p = jnp.zeros((D,), jnp.float32).at[:40].set(bl2)[IDX,]
    br2p = jnp.zeros((D,), jnp.float32).at[:40].set(br2)[IDX,]
    att2p = jnp.zeros((D,), jnp.float32).at[:40].set(att2)

    xl2, xr2 = _combine_transform(np1, den1, bias1, Wl2p, bl2p, Wr2p, br2p)
    np2, dp2 = _edge_pass(src, dst, xl2, xr2, att2p, z64, z1)
    den2 = (dp2[0, :N] + dp2[1, :N] + 1e-16).reshape(N, 1)
    out = _final(np2, den2, bias2)
    return (out, edge_index)
